# Initial kernel scaffold; baseline (speedup 1.0000x reference)
import jax
import jax.numpy as jnp
from jax.experimental import pallas as pl


def kernel(x_game, x_state, edge_game_to_game, edge_state_parent_of_state, edge_game_history_state, edge_game_in_state, W1l, b1l, W1r, W2l, b2l, W2r, W3l, b3l, W3r, W4l, b4l, W4r, Wlin, blin):
    raise NotImplementedError("write your pallas kernel here")


# speedup vs baseline: 3.2406x; 3.2406x over previous
import jax
import jax.numpy as jnp
from jax import lax
from jax.experimental import pallas as pl
from jax.experimental.pallas import tpu as pltpu
from jax.experimental.pallas import tpu_sc as plsc

N_NODE = 10000
D = 128
E = 320000
NPAD = 10240
N_CORES = 2
N_TILES = 16
BATCH = 128

TILE_BATCHES = E // (N_CORES * N_TILES * BATCH) + 1
TILE_E = TILE_BATCHES * BATCH
EPAD = TILE_E * N_CORES * N_TILES

CNT_BATCHES = EPAD // (N_TILES * BATCH)
CNT_TILE_E = CNT_BATCHES * BATCH

ROWS_T = NPAD // N_TILES


def _sc_agg_body(tab, src, dst, psum,
                 acc, sidx, didx, rows, zrow, sem):
    cid = lax.axis_index("c")
    sid = lax.axis_index("s")
    wid = cid * N_TILES + sid
    tbase = sid * ROWS_T

    zero16 = jnp.zeros((16,), jnp.float32)
    for r in range(16):
        for c8 in range(D // 16):
            zrow[r, pl.ds(c8 * 16, 16)] = zero16
    for k in range(ROWS_T // 16):
        pltpu.sync_copy(zrow, acc.at[pl.ds(tbase + k * 16, 16)])
    plsc.subcore_barrier()

    ebase = wid * TILE_E

    @pl.loop(0, TILE_BATCHES)
    def _(j):
        base = ebase + j * BATCH
        pltpu.sync_copy(src.at[pl.ds(base, BATCH)], sidx)
        pltpu.async_copy(tab.at[sidx], rows, sem).wait()
        pltpu.sync_copy(dst.at[pl.ds(base, BATCH)], didx)
        pltpu.sync_copy(rows, acc.at[didx], add=True)

    plsc.subcore_barrier()
    pltpu.sync_copy(acc.at[pl.ds(tbase, ROWS_T)],
                    psum.at[cid, pl.ds(tbase, ROWS_T)])


_sc_agg = pl.kernel(
    _sc_agg_body,
    out_type=[jax.ShapeDtypeStruct((N_CORES, NPAD, D), jnp.float32)],
    mesh=plsc.VectorSubcoreMesh(core_axis_name="c", subcore_axis_name="s"),
    scratch_types=[
        pltpu.VMEM_SHARED((NPAD, D), jnp.float32),
        pltpu.VMEM((BATCH,), jnp.int32),
        pltpu.VMEM((BATCH,), jnp.int32),
        pltpu.VMEM((BATCH, D), jnp.float32),
        pltpu.VMEM((16, D), jnp.float32),
        pltpu.SemaphoreType.DMA,
    ],
)


def _sc_cnt_body(dst, pcnt, acc, didx, ones_v, zrow):
    cid = lax.axis_index("c")
    sid = lax.axis_index("s")
    wid = cid * N_TILES + sid
    tbase = sid * ROWS_T

    zero16 = jnp.zeros((16,), jnp.float32)
    one16 = jnp.ones((16,), jnp.float32)
    for r in range(16):
        for c8 in range(D // 16):
            zrow[r, pl.ds(c8 * 16, 16)] = zero16
    for r in range(BATCH):
        for c8 in range(D // 16):
            ones_v[r, pl.ds(c8 * 16, 16)] = one16
    for k in range(ROWS_T // 16):
        pltpu.sync_copy(zrow, acc.at[pl.ds(tbase + k * 16, 16)])
    plsc.subcore_barrier()

    ebase = wid * TILE_E

    @pl.loop(0, TILE_BATCHES)
    def _(j):
        base = ebase + j * BATCH
        pltpu.sync_copy(dst.at[pl.ds(base, BATCH)], didx)
        pltpu.sync_copy(ones_v, acc.at[didx], add=True)

    plsc.subcore_barrier()
    pltpu.sync_copy(acc.at[pl.ds(tbase, ROWS_T)],
                    pcnt.at[cid, pl.ds(tbase, ROWS_T)])


_sc_cnt = pl.kernel(
    _sc_cnt_body,
    out_type=[jax.ShapeDtypeStruct((N_CORES, NPAD, D), jnp.float32)],
    mesh=plsc.VectorSubcoreMesh(core_axis_name="c", subcore_axis_name="s"),
    scratch_types=[
        pltpu.VMEM_SHARED((NPAD, D), jnp.float32),
        pltpu.VMEM((BATCH,), jnp.int32),
        pltpu.VMEM((BATCH, D), jnp.float32),
        pltpu.VMEM((16, D), jnp.float32),
    ],
)


ROWS_B = 1024
GRID = NPAD // ROWS_B


def _mean(sa, sb, ca, cb):
    return (sa[0] + sb[0]) / jnp.maximum(ca[0] + cb[0], 1.0)


def _tc_a_body(s1a, s1b, c1a, c1b, xg, s2a, s2b, c2a, c2b, xs,
               w1l, b1l, w1r, w2l, b2l, w2r, gx_o, st2_o):
    m1 = _mean(s1a, s1b, c1a, c1b)
    gx = jnp.dot(m1, w1l[...], preferred_element_type=jnp.float32) + b1l[...]
    gx = gx + jnp.dot(xg[...], w1r[...], preferred_element_type=jnp.float32)
    gx_o[...] = jnp.maximum(gx, 0.0)
    m2 = _mean(s2a, s2b, c2a, c2b)
    s2v = jnp.dot(m2, w2l[...], preferred_element_type=jnp.float32) + b2l[...]
    s2v = s2v + jnp.dot(xs[...], w2r[...], preferred_element_type=jnp.float32)
    st2_o[...] = jnp.maximum(s2v, 0.0)


def _tc_b_body(s3a, s3b, c3a, c3b, s4a, s4b, c4a, c4b, st2,
               w3l, b3l, w3r, w4l, b4l, w4r, wlin, blin, out_o):
    m3 = _mean(s3a, s3b, c3a, c3b)
    s3v = jnp.dot(m3, w3l[...], preferred_element_type=jnp.float32) + b3l[...]
    s3v = s3v + jnp.dot(st2[...], w3r[...], preferred_element_type=jnp.float32)
    st3 = jnp.maximum(s3v, 0.0)
    m4 = _mean(s4a, s4b, c4a, c4b)
    s4v = jnp.dot(m4, w4l[...], preferred_element_type=jnp.float32) + b4l[...]
    s4v = s4v + jnp.dot(st3, w4r[...], preferred_element_type=jnp.float32)
    st4 = jnp.maximum(s4v, 0.0)
    out_o[...] = jnp.dot(st4, wlin[...], preferred_element_type=jnp.float32) + blin[...]


def _row_spec(w):
    return pl.BlockSpec((ROWS_B, w), lambda i: (i, 0))


def _part_spec(core):
    return pl.BlockSpec((1, ROWS_B, D), lambda i, _c=core: (_c, i, 0))


def _cnt_spec(core):
    return pl.BlockSpec((1, ROWS_B, 1), lambda i, _c=core: (_c, i, 0))


def _full_spec(shape):
    return pl.BlockSpec(shape, lambda i: (0,) * len(shape))


def _sum_specs():
    return [_part_spec(0), _part_spec(1), _cnt_spec(0), _cnt_spec(1)]


_tc_phase_a = pl.pallas_call(
    _tc_a_body,
    grid=(GRID,),
    in_specs=(
        _sum_specs() + [_row_spec(D)]
        + _sum_specs() + [_row_spec(D)]
        + [_full_spec((D, D)), _full_spec((1, D)), _full_spec((D, D)),
           _full_spec((D, D)), _full_spec((1, D)), _full_spec((D, D))]
    ),
    out_specs=[_row_spec(D), _row_spec(D)],
    out_shape=[
        jax.ShapeDtypeStruct((N_NODE, D), jnp.float32),
        jax.ShapeDtypeStruct((N_NODE, D), jnp.float32),
    ],
)

OUT_W = 64

_tc_phase_b = pl.pallas_call(
    _tc_b_body,
    grid=(GRID,),
    in_specs=(
        _sum_specs() + _sum_specs() + [_row_spec(D)]
        + [_full_spec((D, D)), _full_spec((1, D)), _full_spec((D, D)),
           _full_spec((D, D)), _full_spec((1, D)), _full_spec((D, D)),
           _full_spec((D, OUT_W)), _full_spec((1, OUT_W))]
    ),
    out_specs=[_row_spec(OUT_W)],
    out_shape=[jax.ShapeDtypeStruct((N_NODE, OUT_W), jnp.float32)],
)


def _pad_edges(e):
    src = jnp.concatenate(
        [e[0].astype(jnp.int32), jnp.zeros((EPAD - E,), jnp.int32)])
    dst = jnp.concatenate(
        [e[1].astype(jnp.int32),
         jnp.full((EPAD - E,), N_NODE, jnp.int32)])
    return src, dst


def kernel(x_game, x_state, edge_game_to_game, edge_state_parent_of_state,
           edge_game_history_state, edge_game_in_state,
           W1l, b1l, W1r, W2l, b2l, W2r, W3l, b3l, W3r, W4l, b4l, W4r,
           Wlin, blin):
    src1, dst1 = _pad_edges(edge_game_to_game)
    src2, dst2 = _pad_edges(edge_state_parent_of_state)
    src3, dst3 = _pad_edges(edge_game_history_state)
    src4, dst4 = _pad_edges(edge_game_in_state)

    (c1,) = _sc_cnt(dst1)
    (c2,) = _sc_cnt(dst2)
    (c3,) = _sc_cnt(dst3)
    (c4,) = _sc_cnt(dst4)
    c1, c2, c3, c4 = (c[:, :, :1] for c in (c1, c2, c3, c4))

    (s1,) = _sc_agg(x_game, src1, dst1)
    (s2,) = _sc_agg(x_state, src2, dst2)
    gx, st2 = _tc_phase_a(
        s1, s1, c1, c1, x_game,
        s2, s2, c2, c2, x_state,
        W1l, b1l.reshape(1, D), W1r, W2l, b2l.reshape(1, D), W2r)

    (s3,) = _sc_agg(gx, src3, dst3)
    (s4,) = _sc_agg(gx, src4, dst4)
    (out,) = _tc_phase_b(
        s3, s3, c3, c3, s4, s4, c4, c4, st2,
        W3l, b3l.reshape(1, D), W3r, W4l, b4l.reshape(1, D), W4r,
        Wlin, blin.reshape(1, OUT_W))
    return out


# speedup vs baseline: 3.2410x; 1.0001x over previous
import jax
import jax.numpy as jnp
from jax import lax
from jax.experimental import pallas as pl
from jax.experimental.pallas import tpu as pltpu
from jax.experimental.pallas import tpu_sc as plsc

N_NODE = 10000
D = 128
E = 320000
NPAD = 10240
N_CORES = 2
N_TILES = 16
BATCH = 128

TILE_BATCHES = E // (N_CORES * N_TILES * BATCH) + 1
TILE_E = TILE_BATCHES * BATCH
EPAD = TILE_E * N_CORES * N_TILES

ROWS_T = NPAD // N_TILES


def _sc_agg_body(tab, src, dst, psum,
                 acc, sidx, didx, rows, zrow, sem):
    cid = lax.axis_index("c")
    sid = lax.axis_index("s")
    wid = cid * N_TILES + sid
    tbase = sid * ROWS_T

    zero16 = jnp.zeros((16,), jnp.float32)
    for r in range(16):
        for c8 in range(D // 16):
            zrow[r, pl.ds(c8 * 16, 16)] = zero16
    for k in range(ROWS_T // 16):
        pltpu.sync_copy(zrow, acc.at[pl.ds(tbase + k * 16, 16)])
    plsc.subcore_barrier()

    ebase = wid * TILE_E

    @pl.loop(0, TILE_BATCHES)
    def _(j):
        base = ebase + j * BATCH
        pltpu.sync_copy(src.at[pl.ds(base, BATCH)], sidx)
        pltpu.async_copy(tab.at[sidx], rows, sem).wait()
        pltpu.sync_copy(dst.at[pl.ds(base, BATCH)], didx)
        pltpu.sync_copy(rows, acc.at[didx], add=True)

    plsc.subcore_barrier()
    pltpu.sync_copy(acc.at[pl.ds(tbase, ROWS_T)],
                    psum.at[cid, pl.ds(tbase, ROWS_T)])


_sc_agg = pl.kernel(
    _sc_agg_body,
    out_type=[jax.ShapeDtypeStruct((N_CORES, NPAD, D), jnp.float32)],
    mesh=plsc.VectorSubcoreMesh(core_axis_name="c", subcore_axis_name="s"),
    scratch_types=[
        pltpu.VMEM_SHARED((NPAD, D), jnp.float32),
        pltpu.VMEM((BATCH,), jnp.int32),
        pltpu.VMEM((BATCH,), jnp.int32),
        pltpu.VMEM((BATCH, D), jnp.float32),
        pltpu.VMEM((16, D), jnp.float32),
        pltpu.SemaphoreType.DMA,
    ],
)


def _sc_cnt_body(dst, pcnt, acc, didx, ones_v, zrow):
    cid = lax.axis_index("c")
    sid = lax.axis_index("s")
    wid = cid * N_TILES + sid
    tbase = sid * ROWS_T

    zero16 = jnp.zeros((16,), jnp.float32)
    one16 = jnp.ones((16,), jnp.float32)
    for r in range(16):
        for c8 in range(D // 16):
            zrow[r, pl.ds(c8 * 16, 16)] = zero16
    for r in range(BATCH):
        for c8 in range(D // 16):
            ones_v[r, pl.ds(c8 * 16, 16)] = one16
    for k in range(ROWS_T // 16):
        pltpu.sync_copy(zrow, acc.at[pl.ds(tbase + k * 16, 16)])
    plsc.subcore_barrier()

    ebase = wid * TILE_E

    @pl.loop(0, TILE_BATCHES)
    def _(j):
        base = ebase + j * BATCH
        pltpu.sync_copy(dst.at[pl.ds(base, BATCH)], didx)
        pltpu.sync_copy(ones_v, acc.at[didx], add=True)

    plsc.subcore_barrier()
    pltpu.sync_copy(acc.at[pl.ds(tbase, ROWS_T)],
                    pcnt.at[cid, pl.ds(tbase, ROWS_T)])


_sc_cnt = pl.kernel(
    _sc_cnt_body,
    out_type=[jax.ShapeDtypeStruct((N_CORES, NPAD, D), jnp.float32)],
    mesh=plsc.VectorSubcoreMesh(core_axis_name="c", subcore_axis_name="s"),
    scratch_types=[
        pltpu.VMEM_SHARED((NPAD, D), jnp.float32),
        pltpu.VMEM((BATCH,), jnp.int32),
        pltpu.VMEM((BATCH, D), jnp.float32),
        pltpu.VMEM((16, D), jnp.float32),
    ],
)


ROWS_B = 1024
GRID = NPAD // ROWS_B


def _mean(sa, sb, ca, cb):
    return (sa[0] + sb[0]) / jnp.maximum(ca[0] + cb[0], 1.0)


def _tc_a_body(s1a, s1b, c1a, c1b, xg, s2a, s2b, c2a, c2b, xs,
               w1l, b1l, w1r, w2l, b2l, w2r, gx_o, st2_o):
    m1 = _mean(s1a, s1b, c1a, c1b)
    gx = jnp.dot(m1, w1l[...], preferred_element_type=jnp.float32) + b1l[...]
    gx = gx + jnp.dot(xg[...], w1r[...], preferred_element_type=jnp.float32)
    gx_o[...] = jnp.maximum(gx, 0.0)
    m2 = _mean(s2a, s2b, c2a, c2b)
    s2v = jnp.dot(m2, w2l[...], preferred_element_type=jnp.float32) + b2l[...]
    s2v = s2v + jnp.dot(xs[...], w2r[...], preferred_element_type=jnp.float32)
    st2_o[...] = jnp.maximum(s2v, 0.0)


def _tc_b_body(s3a, s3b, c3a, c3b, s4a, s4b, c4a, c4b, st2,
               w3l, b3l, w3r, w4l, b4l, w4r, wlin, blin, out_o):
    m3 = _mean(s3a, s3b, c3a, c3b)
    s3v = jnp.dot(m3, w3l[...], preferred_element_type=jnp.float32) + b3l[...]
    s3v = s3v + jnp.dot(st2[...], w3r[...], preferred_element_type=jnp.float32)
    st3 = jnp.maximum(s3v, 0.0)
    m4 = _mean(s4a, s4b, c4a, c4b)
    s4v = jnp.dot(m4, w4l[...], preferred_element_type=jnp.float32) + b4l[...]
    s4v = s4v + jnp.dot(st3, w4r[...], preferred_element_type=jnp.float32)
    st4 = jnp.maximum(s4v, 0.0)
    out_o[...] = jnp.dot(st4, wlin[...], preferred_element_type=jnp.float32) + blin[...]


def _row_spec(w):
    return pl.BlockSpec((ROWS_B, w), lambda i: (i, 0))


def _part_spec(core):
    return pl.BlockSpec((1, ROWS_B, D), lambda i, _c=core: (_c, i, 0))


def _cnt_spec(core):
    return pl.BlockSpec((1, ROWS_B, 1), lambda i, _c=core: (_c, i, 0))


def _full_spec(shape):
    return pl.BlockSpec(shape, lambda i: (0,) * len(shape))


def _sum_specs():
    return [_part_spec(0), _part_spec(1), _cnt_spec(0), _cnt_spec(1)]


_tc_phase_a = pl.pallas_call(
    _tc_a_body,
    grid=(GRID,),
    in_specs=(
        _sum_specs() + [_row_spec(D)]
        + _sum_specs() + [_row_spec(D)]
        + [_full_spec((D, D)), _full_spec((1, D)), _full_spec((D, D)),
           _full_spec((D, D)), _full_spec((1, D)), _full_spec((D, D))]
    ),
    out_specs=[_row_spec(D), _row_spec(D)],
    out_shape=[
        jax.ShapeDtypeStruct((N_NODE, D), jnp.float32),
        jax.ShapeDtypeStruct((N_NODE, D), jnp.float32),
    ],
)

OUT_W = 64

_tc_phase_b = pl.pallas_call(
    _tc_b_body,
    grid=(GRID,),
    in_specs=(
        _sum_specs() + _sum_specs() + [_row_spec(D)]
        + [_full_spec((D, D)), _full_spec((1, D)), _full_spec((D, D)),
           _full_spec((D, D)), _full_spec((1, D)), _full_spec((D, D)),
           _full_spec((D, OUT_W)), _full_spec((1, OUT_W))]
    ),
    out_specs=[_row_spec(OUT_W)],
    out_shape=[jax.ShapeDtypeStruct((N_NODE, OUT_W), jnp.float32)],
)


def _pad_edges(e):
    src = jnp.concatenate(
        [e[0].astype(jnp.int32), jnp.zeros((EPAD - E,), jnp.int32)])
    dst = jnp.concatenate(
        [e[1].astype(jnp.int32),
         jnp.full((EPAD - E,), N_NODE, jnp.int32)])
    return src, dst


def kernel(x_game, x_state, edge_game_to_game, edge_state_parent_of_state,
           edge_game_history_state, edge_game_in_state,
           W1l, b1l, W1r, W2l, b2l, W2r, W3l, b3l, W3r, W4l, b4l, W4r,
           Wlin, blin):
    src1, dst1 = _pad_edges(edge_game_to_game)
    src2, dst2 = _pad_edges(edge_state_parent_of_state)
    src3, dst3 = _pad_edges(edge_game_history_state)
    src4, dst4 = _pad_edges(edge_game_in_state)

    (c1,) = _sc_cnt(dst1)
    (c2,) = _sc_cnt(dst2)
    (c3,) = _sc_cnt(dst3)
    (c4,) = _sc_cnt(dst4)
    c1, c2, c3, c4 = (c[:, :, :1] for c in (c1, c2, c3, c4))

    (s1,) = _sc_agg(x_game, src1, dst1)
    (s2,) = _sc_agg(x_state, src2, dst2)
    gx, st2 = _tc_phase_a(
        s1, s1, c1, c1, x_game,
        s2, s2, c2, c2, x_state,
        W1l, b1l.reshape(1, D), W1r, W2l, b2l.reshape(1, D), W2r)

    (s3,) = _sc_agg(gx, src3, dst3)
    (s4,) = _sc_agg(gx, src4, dst4)
    (out,) = _tc_phase_b(
        s3, s3, c3, c3, s4, s4, c4, c4, st2,
        W3l, b3l.reshape(1, D), W3r, W4l, b4l.reshape(1, D), W4r,
        Wlin, blin.reshape(1, OUT_W))
    return out
